# baseline (speedup 1.0000x reference)
import functools

import jax
import jax.numpy as jnp
from jax import lax
from jax.experimental import pallas as pl
from jax.experimental.pallas import tpu as pltpu
from jax.experimental.pallas import tpu_sc as plsc

N_MESH = 10000
P_NODES = 200000
E_PATCH = 400000
E_MESH = 320000
IN_DIM = 32
HID = 128
OUT_FEATS = 16

NW = 32
CN_P = 512
NCHUNK_P = 392
PPAD = NCHUNK_P * CN_P
SLOTS_P = 13
CN_M = 320
NCHUNK_M = 32
MPAD = NCHUNK_M * CN_M
GB = 128
RG = 49
EPAD_EXTRA = 1024


def _agg_build(d_feat, cn, nchunk, slots, weighted):
    npad = nchunk * cn
    mesh = plsc.VectorSubcoreMesh(core_axis_name="c", subcore_axis_name="s")

    @functools.partial(
        pl.kernel,
        out_type=(
            jax.ShapeDtypeStruct((npad, d_feat), jnp.float32),
            jax.ShapeDtypeStruct((npad,), jnp.float32),
        ),
        mesh=mesh,
        compiler_params=pltpu.CompilerParams(needs_layout_passes=False),
        scratch_types=[
            pltpu.VMEM((16,), jnp.int32),
            pltpu.VMEM((16,), jnp.int32),
            pltpu.VMEM((GB,), jnp.int32),
            pltpu.VMEM((GB,), jnp.int32),
            pltpu.VMEM((GB,), jnp.float32),
            pltpu.VMEM((GB, d_feat), jnp.float32),
            pltpu.VMEM((cn, d_feat), jnp.float32),
            pltpu.VMEM((cn,), jnp.float32),
            pltpu.SemaphoreType.DMA,
        ],
    )
    def k(h_hbm, ss_hbm, ds_hbm, ws_hbm, meta_hbm, num_hbm, den_hbm,
          ms8_v, mng_v, src_v, dst_v, w_v, rows_v, acc_v, accd_v, sem):
        cid = lax.axis_index("c")
        sid = lax.axis_index("s")
        wid = sid * 2 + cid
        lanes = lax.iota(jnp.int32, 16)
        zero16 = jnp.zeros((16,), jnp.float32)

        moff = pl.multiple_of(wid * 16, 8)
        pltpu.sync_copy(meta_hbm.at[pl.ds(moff, 16)], ms8_v)
        pltpu.sync_copy(meta_hbm.at[pl.ds(512 + moff, 16)], mng_v)

        def slot_body(slot, _):
            c = wid + 32 * slot
            ssplat = jnp.zeros((16,), jnp.int32) + slot

            @pl.when(c < nchunk)
            def _process():
                lo = c * cn
                hi = lo + cn
                s8 = pl.multiple_of(plsc.load_gather(ms8_v, [ssplat])[0], 8)
                ng = plsc.load_gather(mng_v, [ssplat])[0]

                def zrow(i, _):
                    def zcol(j, _):
                        acc_v[i, pl.ds(j * 16, 16)] = zero16
                        return 0
                    lax.fori_loop(0, d_feat // 16, zcol, 0, unroll=True)
                    return 0
                lax.fori_loop(0, cn, zrow, 0)

                def zden(i, _):
                    accd_v[pl.ds(i * 16, 16)] = zero16
                    return 0
                lax.fori_loop(0, cn // 16, zden, 0)

                def group(g, _):
                    e0 = pl.multiple_of(s8 + g * GB, 8)
                    pltpu.sync_copy(ss_hbm.at[pl.ds(e0, GB)], src_v)
                    pltpu.sync_copy(ds_hbm.at[pl.ds(e0, GB)], dst_v)
                    if weighted:
                        pltpu.sync_copy(ws_hbm.at[pl.ds(e0, GB)], w_v)
                    pltpu.async_copy(h_hbm.at[src_v], rows_v, sem).wait()

                    def batch(b, _):
                        dv = dst_v[pl.ds(b * 16, 16)]
                        m = jnp.logical_and(dv >= lo, dv < hi)
                        if weighted:
                            wv = w_v[pl.ds(b * 16, 16)]
                        else:
                            wv = zero16 + 1.0
                        wm = jnp.where(m, wv, 0.0)
                        dl = jnp.where(m, dv - lo, 0)
                        plsc.addupdate_scatter(accd_v, [dl], wm)
                        rowid = b * 16 + lanes

                        def cstep(cc, _):
                            cs = jnp.zeros((16,), jnp.int32) + cc
                            vals = plsc.load_gather(rows_v, [rowid, cs])
                            plsc.addupdate_scatter(acc_v, [dl, cs], vals * wm)
                            return 0
                        lax.fori_loop(0, d_feat, cstep, 0, unroll=4)
                        return 0
                    lax.fori_loop(0, GB // 16, batch, 0)
                    return 0
                lax.fori_loop(0, ng, group, 0)

                lo8 = pl.multiple_of(lo, 8)
                pltpu.sync_copy(acc_v, num_hbm.at[pl.ds(lo8, cn)])
                pltpu.sync_copy(accd_v, den_hbm.at[pl.ds(lo8, cn)])

            return 0

        lax.fori_loop(0, slots, slot_body, 0)

    return k


def _readout_build():
    mesh = plsc.VectorSubcoreMesh(core_axis_name="c", subcore_axis_name="s")

    @functools.partial(
        pl.kernel,
        out_type=(
            jax.ShapeDtypeStruct((MPAD, HID), jnp.float32),
            jax.ShapeDtypeStruct((MPAD,), jnp.float32),
        ),
        mesh=mesh,
        compiler_params=pltpu.CompilerParams(needs_layout_passes=False),
        scratch_types=[
            pltpu.VMEM((16,), jnp.int32),
            pltpu.VMEM((16,), jnp.int32),
            pltpu.VMEM((GB,), jnp.int32),
            pltpu.VMEM((GB, HID), jnp.float32),
            pltpu.VMEM((CN_M, HID), jnp.float32),
            pltpu.VMEM((CN_M,), jnp.float32),
            pltpu.SemaphoreType.DMA,
        ],
    )
    def k(h_hbm, ids_hbm, meta_hbm, sums_hbm, cnt_hbm,
          ms8_v, mng_v, ids_v, rows_v, acc_v, accd_v, sem):
        cid = lax.axis_index("c")
        sid = lax.axis_index("s")
        wid = sid * 2 + cid
        lanes = lax.iota(jnp.int32, 16)
        zero16 = jnp.zeros((16,), jnp.float32)
        m_lo = pl.multiple_of(wid * CN_M, 8)
        m_hi = m_lo + CN_M
        moff = pl.multiple_of(wid * 16, 8)
        pltpu.sync_copy(meta_hbm.at[pl.ds(moff, 16)], ms8_v)
        pltpu.sync_copy(meta_hbm.at[pl.ds(512 + moff, 16)], mng_v)
        r0 = pl.multiple_of(ms8_v[...][0], 8)
        ng = mng_v[...][0]

        def zrow(i, _):
            def zcol(j, _):
                acc_v[i, pl.ds(j * 16, 16)] = zero16
                return 0
            lax.fori_loop(0, HID // 16, zcol, 0, unroll=True)
            return 0
        lax.fori_loop(0, CN_M, zrow, 0)

        def zden(i, _):
            accd_v[pl.ds(i * 16, 16)] = zero16
            return 0
        lax.fori_loop(0, CN_M // 16, zden, 0)

        def group(g, _):
            e0 = pl.multiple_of(r0 + g * GB, 8)
            pltpu.sync_copy(ids_hbm.at[pl.ds(e0, GB)], ids_v)
            pltpu.sync_copy(h_hbm.at[pl.ds(e0, GB)], rows_v)

            def batch(b, _):
                iv = ids_v[pl.ds(b * 16, 16)]
                m = jnp.logical_and(iv >= m_lo, iv < m_hi)
                wm = jnp.where(m, 1.0, 0.0)
                dl = jnp.where(m, iv - m_lo, 0)
                plsc.addupdate_scatter(accd_v, [dl], wm)
                rowid = b * 16 + lanes

                def cstep(cc, _):
                    cs = jnp.zeros((16,), jnp.int32) + cc
                    vals = plsc.load_gather(rows_v, [rowid, cs])
                    plsc.addupdate_scatter(acc_v, [dl, cs], vals * wm)
                    return 0
                lax.fori_loop(0, HID, cstep, 0, unroll=4)
                return 0
            lax.fori_loop(0, GB // 16, batch, 0)
            return 0
        lax.fori_loop(0, ng, group, 0)

        mlo8 = pl.multiple_of(m_lo, 8)
        pltpu.sync_copy(acc_v, sums_hbm.at[pl.ds(mlo8, CN_M)])
        pltpu.sync_copy(accd_v, cnt_hbm.at[pl.ds(mlo8, CN_M)])

    return k


def _dense(num, den, w, eps, rows_out, blk):
    nb = num.shape[0] // blk
    d_in = num.shape[1]
    d_out = w.shape[1]

    def body(x_ref, d_ref, w_ref, o_ref):
        r = 1.0 / jnp.maximum(d_ref[...], eps)
        x = x_ref[...] * r
        o_ref[...] = jnp.maximum(
            jnp.dot(x, w_ref[...], precision=lax.Precision.HIGHEST,
                    preferred_element_type=jnp.float32), 0.0)

    return pl.pallas_call(
        body,
        grid=(nb,),
        in_specs=[
            pl.BlockSpec((blk, d_in), lambda i: (i, 0)),
            pl.BlockSpec((blk, 1), lambda i: (i, 0)),
            pl.BlockSpec((d_in, d_out), lambda i: (0, 0)),
        ],
        out_specs=pl.BlockSpec((blk, d_out), lambda i: (i, 0)),
        out_shape=jax.ShapeDtypeStruct((rows_out, d_out), jnp.float32),
    )(num, den.reshape(-1, 1), w)


def _matmul_plain(x, w, blk):
    rows = x.shape[0]
    nb = pl.cdiv(rows, blk)
    d_in = x.shape[1]
    d_out = w.shape[1]

    def body(x_ref, w_ref, o_ref):
        o_ref[...] = jnp.dot(x_ref[...], w_ref[...],
                             precision=lax.Precision.HIGHEST,
                             preferred_element_type=jnp.float32)

    return pl.pallas_call(
        body,
        grid=(nb,),
        in_specs=[
            pl.BlockSpec((blk, d_in), lambda i: (i, 0)),
            pl.BlockSpec((d_in, d_out), lambda i: (0, 0)),
        ],
        out_specs=pl.BlockSpec((blk, d_out), lambda i: (i, 0)),
        out_shape=jax.ShapeDtypeStruct((rows, d_out), jnp.float32),
    )(x, w)


def _scale_relu(x, den, eps, blk):
    nb = x.shape[0] // blk
    d = x.shape[1]

    def body(x_ref, d_ref, o_ref):
        r = 1.0 / jnp.maximum(d_ref[...], eps)
        o_ref[...] = jnp.maximum(x_ref[...] * r, 0.0)

    return pl.pallas_call(
        body,
        grid=(nb,),
        in_specs=[
            pl.BlockSpec((blk, d), lambda i: (i, 0)),
            pl.BlockSpec((blk, 1), lambda i: (i, 0)),
        ],
        out_specs=pl.BlockSpec((blk, d), lambda i: (i, 0)),
        out_shape=jax.ShapeDtypeStruct(x.shape, jnp.float32),
    )(x, den.reshape(-1, 1))


def _rowscale(x, cnt, eps, blk):
    nb = x.shape[0] // blk
    d = x.shape[1]

    def body(x_ref, d_ref, o_ref):
        o_ref[...] = x_ref[...] * (1.0 / jnp.maximum(d_ref[...], eps))

    return pl.pallas_call(
        body,
        grid=(nb,),
        in_specs=[
            pl.BlockSpec((blk, d), lambda i: (i, 0)),
            pl.BlockSpec((blk, 1), lambda i: (i, 0)),
        ],
        out_specs=pl.BlockSpec((blk, d), lambda i: (i, 0)),
        out_shape=jax.ShapeDtypeStruct(x.shape, jnp.float32),
    )(x, cnt.reshape(-1, 1))


def _final(g, wc, n_real, blk):
    nb = g.shape[0] // blk
    d = g.shape[1]
    d_out = wc.shape[1]

    def body(x_ref, w_ref, o_ref):
        i = pl.program_id(0)

        @pl.when(i == 0)
        def _():
            o_ref[...] = jnp.zeros_like(o_ref)

        s = jnp.sum(x_ref[...], axis=0, keepdims=True) * (1.0 / n_real)
        o_ref[...] += jnp.dot(s, w_ref[...], precision=lax.Precision.HIGHEST,
                               preferred_element_type=jnp.float32)

    return pl.pallas_call(
        body,
        grid=(nb,),
        in_specs=[
            pl.BlockSpec((blk, d), lambda i: (i, 0)),
            pl.BlockSpec((d, d_out), lambda i: (0, 0)),
        ],
        out_specs=pl.BlockSpec((1, d_out), lambda i: (0, 0)),
        out_shape=jax.ShapeDtypeStruct((1, d_out), jnp.float32),
    )(g, wc)


def _edge_meta(dst_sorted, cn, nchunk, slots):
    bounds = jnp.arange(nchunk + 1, dtype=jnp.int32) * cn
    pos = jnp.searchsorted(dst_sorted, bounds).astype(jnp.int32)
    starts = pos[:nchunk]
    ends = pos[1:]
    s8 = (starts // 8) * 8
    ng = (ends - s8 + (GB - 1)) // GB
    npad = slots * 32
    s8 = jnp.pad(s8, (0, npad - nchunk))
    ng = jnp.pad(ng, (0, npad - nchunk))
    s8 = jnp.pad(s8.reshape(slots, 32).T, ((0, 0), (0, 16 - slots)))
    ng = jnp.pad(ng.reshape(slots, 32).T, ((0, 0), (0, 16 - slots)))
    return jnp.concatenate([s8.reshape(-1), ng.reshape(-1)])


def kernel(patch_x, patch_edge_index, patch_edge_weight, patch_segment_ids,
           mesh_edge_index, W1, W2, W3, Wm1, Wm2, Wc):
    p_dst, p_src, p_w = lax.sort(
        (patch_edge_index[1], patch_edge_index[0], patch_edge_weight),
        num_keys=1)
    m_dst, m_src = lax.sort((mesh_edge_index[1], mesh_edge_index[0]),
                            num_keys=1)
    big = jnp.int32(2**30)
    p_src_p = jnp.pad(p_src, (0, EPAD_EXTRA))
    p_dst_p = jnp.pad(p_dst, (0, EPAD_EXTRA), constant_values=big)
    p_w_p = jnp.pad(p_w, (0, EPAD_EXTRA))
    m_src_p = jnp.pad(m_src, (0, EPAD_EXTRA))
    m_dst_p = jnp.pad(m_dst, (0, EPAD_EXTRA), constant_values=big)
    m_w_p = jnp.zeros((1,), jnp.float32)

    meta_p = _edge_meta(p_dst, CN_P, NCHUNK_P, SLOTS_P)
    meta_m = _edge_meta(m_dst, CN_M, NCHUNK_M, 1)
    ids_pad = jnp.pad(patch_segment_ids.astype(jnp.int32),
                      (0, PPAD - P_NODES), constant_values=MPAD)

    agg = _agg_build(HID, CN_P, NCHUNK_P, SLOTS_P, True)
    t = _matmul_plain(patch_x, W1, CN_P)
    num1, den1 = agg(t, p_src_p, p_dst_p, p_w_p, meta_p)
    h = _scale_relu(num1, den1, 1e-6, CN_P)

    num2, den2 = agg(h, p_src_p, p_dst_p, p_w_p, meta_p)
    h = _dense(num2, den2, W2, 1e-6, PPAD, CN_P)
    num3, den3 = agg(h, p_src_p, p_dst_p, p_w_p, meta_p)
    h = _dense(num3, den3, W3, 1e-6, PPAD, CN_P)

    meta_ro = _edge_meta(patch_segment_ids.astype(jnp.int32), CN_M, NCHUNK_M, 1)
    ro = _readout_build()
    sums, cnt = ro(h, ids_pad, meta_ro)
    readouts = _rowscale(sums, cnt, 1.0, CN_M)

    aggm = _agg_build(HID, CN_M, NCHUNK_M, 1, False)
    nm1, dm1 = aggm(readouts, m_src_p, m_dst_p, m_w_p, meta_m)
    g = _dense(nm1, dm1, Wm1, 1.0, MPAD, CN_M)
    nm2, dm2 = aggm(g, m_src_p, m_dst_p, m_w_p, meta_m)
    g = _dense(nm2, dm2, Wm2, 1.0, MPAD, CN_M)

    out = _final(g, Wc, float(N_MESH), CN_M)
    return out.reshape(OUT_FEATS)


# speedup vs baseline: 2.9593x; 2.9593x over previous
import functools

import jax
import jax.numpy as jnp
from jax import lax
from jax.experimental import pallas as pl
from jax.experimental.pallas import tpu as pltpu
from jax.experimental.pallas import tpu_sc as plsc

N_MESH = 10000
P_NODES = 200000
E_PATCH = 400000
E_MESH = 320000
IN_DIM = 32
HID = 128
OUT_FEATS = 16

NW = 32
CN_P = 512
NCHUNK_P = 392
PPAD = NCHUNK_P * CN_P
SLOTS_P = 13
CN_M = 320
NCHUNK_M = 32
MPAD = NCHUNK_M * CN_M
GB = 128
RG = 49
EPAD_EXTRA = 1024


def _agg_build(d_feat, cn, nchunk, slots, weighted):
    npad = nchunk * cn
    mesh = plsc.VectorSubcoreMesh(core_axis_name="c", subcore_axis_name="s")

    @functools.partial(
        pl.kernel,
        out_type=(
            jax.ShapeDtypeStruct((npad, d_feat), jnp.float32),
            jax.ShapeDtypeStruct((npad,), jnp.float32),
        ),
        mesh=mesh,
        compiler_params=pltpu.CompilerParams(needs_layout_passes=False),
        scratch_types=[
            pltpu.VMEM((16,), jnp.int32),
            pltpu.VMEM((16,), jnp.int32),
            pltpu.VMEM((GB,), jnp.int32),
            pltpu.VMEM((GB,), jnp.int32),
            pltpu.VMEM((GB,), jnp.float32),
            pltpu.VMEM((GB, d_feat), jnp.float32),
            pltpu.VMEM((cn, d_feat), jnp.float32),
            pltpu.VMEM((cn,), jnp.float32),
            pltpu.SemaphoreType.DMA,
        ],
    )
    def k(h_hbm, ss_hbm, ds_hbm, ws_hbm, meta_hbm, num_hbm, den_hbm,
          ms8_v, mng_v, src_v, dst_v, w_v, rows_v, acc_v, accd_v, sem):
        cid = lax.axis_index("c")
        sid = lax.axis_index("s")
        wid = sid * 2 + cid
        lanes = lax.iota(jnp.int32, 16)
        zero16 = jnp.zeros((16,), jnp.float32)

        moff = pl.multiple_of(wid * 16, 8)
        pltpu.sync_copy(meta_hbm.at[pl.ds(moff, 16)], ms8_v)
        pltpu.sync_copy(meta_hbm.at[pl.ds(512 + moff, 16)], mng_v)

        def slot_body(slot, _):
            c = wid + 32 * slot
            ssplat = jnp.zeros((16,), jnp.int32) + slot

            @pl.when(c < nchunk)
            def _process():
                lo = c * cn
                hi = lo + cn
                s8 = pl.multiple_of(plsc.load_gather(ms8_v, [ssplat])[0], 8)
                ng = plsc.load_gather(mng_v, [ssplat])[0]

                def zrow(i, _):
                    def zcol(j, _):
                        acc_v[i, pl.ds(j * 16, 16)] = zero16
                        return 0
                    lax.fori_loop(0, d_feat // 16, zcol, 0, unroll=True)
                    return 0
                lax.fori_loop(0, cn, zrow, 0)

                def zden(i, _):
                    accd_v[pl.ds(i * 16, 16)] = zero16
                    return 0
                lax.fori_loop(0, cn // 16, zden, 0)

                def group(g, _):
                    e0 = pl.multiple_of(s8 + g * GB, 8)
                    pltpu.sync_copy(ss_hbm.at[pl.ds(e0, GB)], src_v)
                    gcopy = pltpu.async_copy(h_hbm.at[src_v], rows_v, sem)
                    pltpu.sync_copy(ds_hbm.at[pl.ds(e0, GB)], dst_v)
                    if weighted:
                        pltpu.sync_copy(ws_hbm.at[pl.ds(e0, GB)], w_v)
                    gcopy.wait()

                    def batch(b, _):
                        dv = dst_v[pl.ds(b * 16, 16)]
                        m = jnp.logical_and(dv >= lo, dv < hi)
                        if weighted:
                            wv = w_v[pl.ds(b * 16, 16)]
                        else:
                            wv = zero16 + 1.0
                        wm = jnp.where(m, wv, 0.0)
                        dl = jnp.where(m, dv - lo, 0)
                        plsc.addupdate_scatter(accd_v, [dl], wm)

                        for j in range(16):
                            r = dl[j]
                            wj = wm[j]
                            e = b * 16 + j
                            for kk in range(d_feat // 16):
                                x = rows_v[e, pl.ds(kk * 16, 16)]
                                plsc.addupdate(
                                    acc_v.at[r, pl.ds(kk * 16, 16)], x * wj)
                        return 0
                    lax.fori_loop(0, GB // 16, batch, 0)
                    return 0
                lax.fori_loop(0, ng, group, 0)

                lo8 = pl.multiple_of(lo, 8)
                pltpu.sync_copy(acc_v, num_hbm.at[pl.ds(lo8, cn)])
                pltpu.sync_copy(accd_v, den_hbm.at[pl.ds(lo8, cn)])

            return 0

        lax.fori_loop(0, slots, slot_body, 0)

    return k


def _readout_build():
    mesh = plsc.VectorSubcoreMesh(core_axis_name="c", subcore_axis_name="s")

    @functools.partial(
        pl.kernel,
        out_type=(
            jax.ShapeDtypeStruct((MPAD, HID), jnp.float32),
            jax.ShapeDtypeStruct((MPAD,), jnp.float32),
        ),
        mesh=mesh,
        compiler_params=pltpu.CompilerParams(needs_layout_passes=False),
        scratch_types=[
            pltpu.VMEM((16,), jnp.int32),
            pltpu.VMEM((16,), jnp.int32),
            pltpu.VMEM((GB,), jnp.int32),
            pltpu.VMEM((GB, HID), jnp.float32),
            pltpu.VMEM((CN_M, HID), jnp.float32),
            pltpu.VMEM((CN_M,), jnp.float32),
            pltpu.SemaphoreType.DMA,
        ],
    )
    def k(h_hbm, ids_hbm, meta_hbm, sums_hbm, cnt_hbm,
          ms8_v, mng_v, ids_v, rows_v, acc_v, accd_v, sem):
        cid = lax.axis_index("c")
        sid = lax.axis_index("s")
        wid = sid * 2 + cid
        lanes = lax.iota(jnp.int32, 16)
        zero16 = jnp.zeros((16,), jnp.float32)
        m_lo = pl.multiple_of(wid * CN_M, 8)
        m_hi = m_lo + CN_M
        moff = pl.multiple_of(wid * 16, 8)
        pltpu.sync_copy(meta_hbm.at[pl.ds(moff, 16)], ms8_v)
        pltpu.sync_copy(meta_hbm.at[pl.ds(512 + moff, 16)], mng_v)
        r0 = pl.multiple_of(ms8_v[...][0], 8)
        ng = mng_v[...][0]

        def zrow(i, _):
            def zcol(j, _):
                acc_v[i, pl.ds(j * 16, 16)] = zero16
                return 0
            lax.fori_loop(0, HID // 16, zcol, 0, unroll=True)
            return 0
        lax.fori_loop(0, CN_M, zrow, 0)

        def zden(i, _):
            accd_v[pl.ds(i * 16, 16)] = zero16
            return 0
        lax.fori_loop(0, CN_M // 16, zden, 0)

        def group(g, _):
            e0 = pl.multiple_of(r0 + g * GB, 8)
            pltpu.sync_copy(ids_hbm.at[pl.ds(e0, GB)], ids_v)
            pltpu.sync_copy(h_hbm.at[pl.ds(e0, GB)], rows_v)

            def batch(b, _):
                iv = ids_v[pl.ds(b * 16, 16)]
                m = jnp.logical_and(iv >= m_lo, iv < m_hi)
                wm = jnp.where(m, 1.0, 0.0)
                dl = jnp.where(m, iv - m_lo, 0)
                plsc.addupdate_scatter(accd_v, [dl], wm)

                for j in range(16):
                    r = dl[j]
                    wj = wm[j]
                    e = b * 16 + j
                    for kk in range(HID // 16):
                        x = rows_v[e, pl.ds(kk * 16, 16)]
                        plsc.addupdate(acc_v.at[r, pl.ds(kk * 16, 16)], x * wj)
                return 0
            lax.fori_loop(0, GB // 16, batch, 0)
            return 0
        lax.fori_loop(0, ng, group, 0)

        mlo8 = pl.multiple_of(m_lo, 8)
        pltpu.sync_copy(acc_v, sums_hbm.at[pl.ds(mlo8, CN_M)])
        pltpu.sync_copy(accd_v, cnt_hbm.at[pl.ds(mlo8, CN_M)])

    return k


def _dense(num, den, w, eps, rows_out, blk):
    nb = num.shape[0] // blk
    d_in = num.shape[1]
    d_out = w.shape[1]

    def body(x_ref, d_ref, w_ref, o_ref):
        r = 1.0 / jnp.maximum(d_ref[...], eps)
        x = x_ref[...] * r
        o_ref[...] = jnp.maximum(
            jnp.dot(x, w_ref[...], precision=lax.Precision.HIGHEST,
                    preferred_element_type=jnp.float32), 0.0)

    return pl.pallas_call(
        body,
        grid=(nb,),
        in_specs=[
            pl.BlockSpec((blk, d_in), lambda i: (i, 0)),
            pl.BlockSpec((blk, 1), lambda i: (i, 0)),
            pl.BlockSpec((d_in, d_out), lambda i: (0, 0)),
        ],
        out_specs=pl.BlockSpec((blk, d_out), lambda i: (i, 0)),
        out_shape=jax.ShapeDtypeStruct((rows_out, d_out), jnp.float32),
    )(num, den.reshape(-1, 1), w)


def _matmul_plain(x, w, blk):
    rows = x.shape[0]
    nb = pl.cdiv(rows, blk)
    d_in = x.shape[1]
    d_out = w.shape[1]

    def body(x_ref, w_ref, o_ref):
        o_ref[...] = jnp.dot(x_ref[...], w_ref[...],
                             precision=lax.Precision.HIGHEST,
                             preferred_element_type=jnp.float32)

    return pl.pallas_call(
        body,
        grid=(nb,),
        in_specs=[
            pl.BlockSpec((blk, d_in), lambda i: (i, 0)),
            pl.BlockSpec((d_in, d_out), lambda i: (0, 0)),
        ],
        out_specs=pl.BlockSpec((blk, d_out), lambda i: (i, 0)),
        out_shape=jax.ShapeDtypeStruct((rows, d_out), jnp.float32),
    )(x, w)


def _scale_relu(x, den, eps, blk):
    nb = x.shape[0] // blk
    d = x.shape[1]

    def body(x_ref, d_ref, o_ref):
        r = 1.0 / jnp.maximum(d_ref[...], eps)
        o_ref[...] = jnp.maximum(x_ref[...] * r, 0.0)

    return pl.pallas_call(
        body,
        grid=(nb,),
        in_specs=[
            pl.BlockSpec((blk, d), lambda i: (i, 0)),
            pl.BlockSpec((blk, 1), lambda i: (i, 0)),
        ],
        out_specs=pl.BlockSpec((blk, d), lambda i: (i, 0)),
        out_shape=jax.ShapeDtypeStruct(x.shape, jnp.float32),
    )(x, den.reshape(-1, 1))


def _rowscale(x, cnt, eps, blk):
    nb = x.shape[0] // blk
    d = x.shape[1]

    def body(x_ref, d_ref, o_ref):
        o_ref[...] = x_ref[...] * (1.0 / jnp.maximum(d_ref[...], eps))

    return pl.pallas_call(
        body,
        grid=(nb,),
        in_specs=[
            pl.BlockSpec((blk, d), lambda i: (i, 0)),
            pl.BlockSpec((blk, 1), lambda i: (i, 0)),
        ],
        out_specs=pl.BlockSpec((blk, d), lambda i: (i, 0)),
        out_shape=jax.ShapeDtypeStruct(x.shape, jnp.float32),
    )(x, cnt.reshape(-1, 1))


def _final(g, wc, n_real, blk):
    nb = g.shape[0] // blk
    d = g.shape[1]
    d_out = wc.shape[1]

    def body(x_ref, w_ref, o_ref):
        i = pl.program_id(0)

        @pl.when(i == 0)
        def _():
            o_ref[...] = jnp.zeros_like(o_ref)

        s = jnp.sum(x_ref[...], axis=0, keepdims=True) * (1.0 / n_real)
        o_ref[...] += jnp.dot(s, w_ref[...], precision=lax.Precision.HIGHEST,
                               preferred_element_type=jnp.float32)

    return pl.pallas_call(
        body,
        grid=(nb,),
        in_specs=[
            pl.BlockSpec((blk, d), lambda i: (i, 0)),
            pl.BlockSpec((d, d_out), lambda i: (0, 0)),
        ],
        out_specs=pl.BlockSpec((1, d_out), lambda i: (0, 0)),
        out_shape=jax.ShapeDtypeStruct((1, d_out), jnp.float32),
    )(g, wc)


def _edge_meta(dst_sorted, cn, nchunk, slots):
    bounds = jnp.arange(nchunk + 1, dtype=jnp.int32) * cn
    pos = jnp.searchsorted(dst_sorted, bounds).astype(jnp.int32)
    starts = pos[:nchunk]
    ends = pos[1:]
    s8 = (starts // 8) * 8
    ng = (ends - s8 + (GB - 1)) // GB
    npad = slots * 32
    s8 = jnp.pad(s8, (0, npad - nchunk))
    ng = jnp.pad(ng, (0, npad - nchunk))
    s8 = jnp.pad(s8.reshape(slots, 32).T, ((0, 0), (0, 16 - slots)))
    ng = jnp.pad(ng.reshape(slots, 32).T, ((0, 0), (0, 16 - slots)))
    return jnp.concatenate([s8.reshape(-1), ng.reshape(-1)])


def kernel(patch_x, patch_edge_index, patch_edge_weight, patch_segment_ids,
           mesh_edge_index, W1, W2, W3, Wm1, Wm2, Wc):
    p_dst, p_src, p_w = lax.sort(
        (patch_edge_index[1], patch_edge_index[0], patch_edge_weight),
        num_keys=1)
    m_dst, m_src = lax.sort((mesh_edge_index[1], mesh_edge_index[0]),
                            num_keys=1)
    big = jnp.int32(2**30)
    p_src_p = jnp.pad(p_src, (0, EPAD_EXTRA))
    p_dst_p = jnp.pad(p_dst, (0, EPAD_EXTRA), constant_values=big)
    p_w_p = jnp.pad(p_w, (0, EPAD_EXTRA))
    m_src_p = jnp.pad(m_src, (0, EPAD_EXTRA))
    m_dst_p = jnp.pad(m_dst, (0, EPAD_EXTRA), constant_values=big)
    m_w_p = jnp.zeros((1,), jnp.float32)

    meta_p = _edge_meta(p_dst, CN_P, NCHUNK_P, SLOTS_P)
    meta_m = _edge_meta(m_dst, CN_M, NCHUNK_M, 1)
    ids_pad = jnp.pad(patch_segment_ids.astype(jnp.int32),
                      (0, PPAD - P_NODES), constant_values=MPAD)

    agg = _agg_build(HID, CN_P, NCHUNK_P, SLOTS_P, True)
    t = _matmul_plain(patch_x, W1, CN_P)
    num1, den1 = agg(t, p_src_p, p_dst_p, p_w_p, meta_p)
    h = _scale_relu(num1, den1, 1e-6, CN_P)

    num2, den2 = agg(h, p_src_p, p_dst_p, p_w_p, meta_p)
    h = _dense(num2, den2, W2, 1e-6, PPAD, CN_P)
    num3, den3 = agg(h, p_src_p, p_dst_p, p_w_p, meta_p)
    h = _dense(num3, den3, W3, 1e-6, PPAD, CN_P)

    meta_ro = _edge_meta(patch_segment_ids.astype(jnp.int32), CN_M, NCHUNK_M, 1)
    ro = _readout_build()
    sums, cnt = ro(h, ids_pad, meta_ro)
    readouts = _rowscale(sums, cnt, 1.0, CN_M)

    aggm = _agg_build(HID, CN_M, NCHUNK_M, 1, False)
    nm1, dm1 = aggm(readouts, m_src_p, m_dst_p, m_w_p, meta_m)
    g = _dense(nm1, dm1, Wm1, 1.0, MPAD, CN_M)
    nm2, dm2 = aggm(g, m_src_p, m_dst_p, m_w_p, meta_m)
    g = _dense(nm2, dm2, Wm2, 1.0, MPAD, CN_M)

    out = _final(g, Wc, float(N_MESH), CN_M)
    return out.reshape(OUT_FEATS)


# speedup vs baseline: 3.0865x; 1.0430x over previous
import functools

import jax
import jax.numpy as jnp
from jax import lax
from jax.experimental import pallas as pl
from jax.experimental.pallas import tpu as pltpu
from jax.experimental.pallas import tpu_sc as plsc

N_MESH = 10000
P_NODES = 200000
E_PATCH = 400000
E_MESH = 320000
IN_DIM = 32
HID = 128
OUT_FEATS = 16

NW = 32
CN_P = 512
NCHUNK_P = 392
PPAD = NCHUNK_P * CN_P
SLOTS_P = 13
CN_M = 320
NCHUNK_M = 32
MPAD = NCHUNK_M * CN_M
GB = 128
RG = 49
EPAD_EXTRA = 1024


def _agg_build(d_feat, cn, nchunk, slots, weighted):
    npad = nchunk * cn
    mesh = plsc.VectorSubcoreMesh(core_axis_name="c", subcore_axis_name="s")

    @functools.partial(
        pl.kernel,
        out_type=(
            jax.ShapeDtypeStruct((npad, d_feat), jnp.float32),
            jax.ShapeDtypeStruct((npad,), jnp.float32),
        ),
        mesh=mesh,
        compiler_params=pltpu.CompilerParams(needs_layout_passes=False),
        scratch_types=[
            pltpu.VMEM((16,), jnp.int32),
            pltpu.VMEM((16,), jnp.int32),
            pltpu.VMEM((2, GB), jnp.int32),
            pltpu.VMEM((2, GB), jnp.int32),
            pltpu.VMEM((2, GB), jnp.float32),
            pltpu.VMEM((2, GB, d_feat), jnp.float32),
            pltpu.VMEM((cn, d_feat), jnp.float32),
            pltpu.VMEM((cn,), jnp.float32),
            pltpu.SemaphoreType.DMA,
            pltpu.SemaphoreType.DMA,
        ],
    )
    def k(h_hbm, ss_hbm, ds_hbm, ws_hbm, meta_hbm, num_hbm, den_hbm,
          ms8_v, mng_v, src_v, dst_v, w_v, rows_v, acc_v, accd_v, sem0, sem1):
        cid = lax.axis_index("c")
        sid = lax.axis_index("s")
        wid = sid * 2 + cid
        lanes = lax.iota(jnp.int32, 16)
        zero16 = jnp.zeros((16,), jnp.float32)

        moff = pl.multiple_of(wid * 16, 8)
        pltpu.sync_copy(meta_hbm.at[pl.ds(moff, 16)], ms8_v)
        pltpu.sync_copy(meta_hbm.at[pl.ds(512 + moff, 16)], mng_v)

        def slot_body(slot, _):
            c = wid + 32 * slot
            ssplat = jnp.zeros((16,), jnp.int32) + slot

            @pl.when(c < nchunk)
            def _process():
                lo = c * cn
                hi = lo + cn
                s8 = pl.multiple_of(plsc.load_gather(ms8_v, [ssplat])[0], 8)
                ng = plsc.load_gather(mng_v, [ssplat])[0]

                def zrow(i, _):
                    def zcol(j, _):
                        acc_v[i, pl.ds(j * 16, 16)] = zero16
                        return 0
                    lax.fori_loop(0, d_feat // 16, zcol, 0, unroll=True)
                    return 0
                lax.fori_loop(0, cn, zrow, 0)

                def zden(i, _):
                    accd_v[pl.ds(i * 16, 16)] = zero16
                    return 0
                lax.fori_loop(0, cn // 16, zden, 0)

                sems = (sem0, sem1)

                def issue(g, p):
                    e0 = pl.multiple_of(s8 + g * GB, 8)
                    pltpu.sync_copy(ss_hbm.at[pl.ds(e0, GB)], src_v.at[p])
                    pltpu.async_copy(h_hbm.at[src_v.at[p]], rows_v.at[p],
                                     sems[p])
                    pltpu.sync_copy(ds_hbm.at[pl.ds(e0, GB)], dst_v.at[p])
                    if weighted:
                        pltpu.sync_copy(ws_hbm.at[pl.ds(e0, GB)], w_v.at[p])

                def compute(p):
                    pltpu.make_async_copy(h_hbm.at[src_v.at[p]],
                                          rows_v.at[p], sems[p]).wait()

                    def batch(b, _):
                        dv = dst_v[p, pl.ds(b * 16, 16)]
                        m = jnp.logical_and(dv >= lo, dv < hi)
                        if weighted:
                            wv = w_v[p, pl.ds(b * 16, 16)]
                        else:
                            wv = zero16 + 1.0
                        wm = jnp.where(m, wv, 0.0)
                        dl = jnp.where(m, dv - lo, 0)
                        plsc.addupdate_scatter(accd_v, [dl], wm)

                        for j in range(16):
                            r = dl[j]
                            wj = wm[j]
                            e = b * 16 + j
                            for kk in range(d_feat // 16):
                                x = rows_v[p, e, pl.ds(kk * 16, 16)]
                                plsc.addupdate(
                                    acc_v.at[r, pl.ds(kk * 16, 16)], x * wj)
                        return 0
                    lax.fori_loop(0, GB // 16, batch, 0)

                @pl.when(ng > 0)
                def _prime():
                    issue(0, 0)

                def gpair(gp, _):
                    g0 = gp * 2

                    @pl.when(g0 + 1 < ng)
                    def _():
                        issue(g0 + 1, 1)
                    compute(0)

                    @pl.when(g0 + 1 < ng)
                    def _():
                        @pl.when(g0 + 2 < ng)
                        def _():
                            issue(g0 + 2, 0)
                        compute(1)
                    return 0
                lax.fori_loop(0, (ng + 1) // 2, gpair, 0)

                lo8 = pl.multiple_of(lo, 8)
                pltpu.sync_copy(acc_v, num_hbm.at[pl.ds(lo8, cn)])
                pltpu.sync_copy(accd_v, den_hbm.at[pl.ds(lo8, cn)])

            return 0

        lax.fori_loop(0, slots, slot_body, 0)

    return k


def _readout_build():
    mesh = plsc.VectorSubcoreMesh(core_axis_name="c", subcore_axis_name="s")

    @functools.partial(
        pl.kernel,
        out_type=(
            jax.ShapeDtypeStruct((MPAD, HID), jnp.float32),
            jax.ShapeDtypeStruct((MPAD,), jnp.float32),
        ),
        mesh=mesh,
        compiler_params=pltpu.CompilerParams(needs_layout_passes=False),
        scratch_types=[
            pltpu.VMEM((16,), jnp.int32),
            pltpu.VMEM((16,), jnp.int32),
            pltpu.VMEM((2, GB), jnp.int32),
            pltpu.VMEM((2, GB, HID), jnp.float32),
            pltpu.VMEM((CN_M, HID), jnp.float32),
            pltpu.VMEM((CN_M,), jnp.float32),
            pltpu.SemaphoreType.DMA,
            pltpu.SemaphoreType.DMA,
        ],
    )
    def k(h_hbm, ids_hbm, meta_hbm, sums_hbm, cnt_hbm,
          ms8_v, mng_v, ids_v, rows_v, acc_v, accd_v, sem0, sem1):
        cid = lax.axis_index("c")
        sid = lax.axis_index("s")
        wid = sid * 2 + cid
        lanes = lax.iota(jnp.int32, 16)
        zero16 = jnp.zeros((16,), jnp.float32)
        m_lo = pl.multiple_of(wid * CN_M, 8)
        m_hi = m_lo + CN_M
        moff = pl.multiple_of(wid * 16, 8)
        pltpu.sync_copy(meta_hbm.at[pl.ds(moff, 16)], ms8_v)
        pltpu.sync_copy(meta_hbm.at[pl.ds(512 + moff, 16)], mng_v)
        r0 = pl.multiple_of(ms8_v[...][0], 8)
        ng = mng_v[...][0]

        def zrow(i, _):
            def zcol(j, _):
                acc_v[i, pl.ds(j * 16, 16)] = zero16
                return 0
            lax.fori_loop(0, HID // 16, zcol, 0, unroll=True)
            return 0
        lax.fori_loop(0, CN_M, zrow, 0)

        def zden(i, _):
            accd_v[pl.ds(i * 16, 16)] = zero16
            return 0
        lax.fori_loop(0, CN_M // 16, zden, 0)

        sems = (sem0, sem1)

        def issue(g, p):
            e0 = pl.multiple_of(r0 + g * GB, 8)
            pltpu.async_copy(h_hbm.at[pl.ds(e0, GB)], rows_v.at[p], sems[p])
            pltpu.sync_copy(ids_hbm.at[pl.ds(e0, GB)], ids_v.at[p])

        def compute(g, p):
            e0 = pl.multiple_of(r0 + g * GB, 8)
            pltpu.make_async_copy(h_hbm.at[pl.ds(e0, GB)], rows_v.at[p],
                                  sems[p]).wait()

            def batch(b, _):
                iv = ids_v[p, pl.ds(b * 16, 16)]
                m = jnp.logical_and(iv >= m_lo, iv < m_hi)
                wm = jnp.where(m, 1.0, 0.0)
                dl = jnp.where(m, iv - m_lo, 0)
                plsc.addupdate_scatter(accd_v, [dl], wm)

                for j in range(16):
                    r = dl[j]
                    wj = wm[j]
                    e = b * 16 + j
                    for kk in range(HID // 16):
                        x = rows_v[p, e, pl.ds(kk * 16, 16)]
                        plsc.addupdate(acc_v.at[r, pl.ds(kk * 16, 16)], x * wj)
                return 0
            lax.fori_loop(0, GB // 16, batch, 0)

        @pl.when(ng > 0)
        def _prime():
            issue(0, 0)

        def gpair(gp, _):
            g0 = gp * 2

            @pl.when(g0 + 1 < ng)
            def _():
                issue(g0 + 1, 1)
            compute(g0, 0)

            @pl.when(g0 + 1 < ng)
            def _():
                @pl.when(g0 + 2 < ng)
                def _():
                    issue(g0 + 2, 0)
                compute(g0 + 1, 1)
            return 0
        lax.fori_loop(0, (ng + 1) // 2, gpair, 0)

        mlo8 = pl.multiple_of(m_lo, 8)
        pltpu.sync_copy(acc_v, sums_hbm.at[pl.ds(mlo8, CN_M)])
        pltpu.sync_copy(accd_v, cnt_hbm.at[pl.ds(mlo8, CN_M)])

    return k


def _dense(num, den, w, eps, rows_out, blk):
    nb = num.shape[0] // blk
    d_in = num.shape[1]
    d_out = w.shape[1]

    def body(x_ref, d_ref, w_ref, o_ref):
        r = 1.0 / jnp.maximum(d_ref[...], eps)
        x = x_ref[...] * r
        o_ref[...] = jnp.maximum(
            jnp.dot(x, w_ref[...], precision=lax.Precision.HIGHEST,
                    preferred_element_type=jnp.float32), 0.0)

    return pl.pallas_call(
        body,
        grid=(nb,),
        in_specs=[
            pl.BlockSpec((blk, d_in), lambda i: (i, 0)),
            pl.BlockSpec((blk, 1), lambda i: (i, 0)),
            pl.BlockSpec((d_in, d_out), lambda i: (0, 0)),
        ],
        out_specs=pl.BlockSpec((blk, d_out), lambda i: (i, 0)),
        out_shape=jax.ShapeDtypeStruct((rows_out, d_out), jnp.float32),
    )(num, den.reshape(-1, 1), w)


def _matmul_plain(x, w, blk):
    rows = x.shape[0]
    nb = pl.cdiv(rows, blk)
    d_in = x.shape[1]
    d_out = w.shape[1]

    def body(x_ref, w_ref, o_ref):
        o_ref[...] = jnp.dot(x_ref[...], w_ref[...],
                             precision=lax.Precision.HIGHEST,
                             preferred_element_type=jnp.float32)

    return pl.pallas_call(
        body,
        grid=(nb,),
        in_specs=[
            pl.BlockSpec((blk, d_in), lambda i: (i, 0)),
            pl.BlockSpec((d_in, d_out), lambda i: (0, 0)),
        ],
        out_specs=pl.BlockSpec((blk, d_out), lambda i: (i, 0)),
        out_shape=jax.ShapeDtypeStruct((rows, d_out), jnp.float32),
    )(x, w)


def _scale_relu(x, den, eps, blk):
    nb = x.shape[0] // blk
    d = x.shape[1]

    def body(x_ref, d_ref, o_ref):
        r = 1.0 / jnp.maximum(d_ref[...], eps)
        o_ref[...] = jnp.maximum(x_ref[...] * r, 0.0)

    return pl.pallas_call(
        body,
        grid=(nb,),
        in_specs=[
            pl.BlockSpec((blk, d), lambda i: (i, 0)),
            pl.BlockSpec((blk, 1), lambda i: (i, 0)),
        ],
        out_specs=pl.BlockSpec((blk, d), lambda i: (i, 0)),
        out_shape=jax.ShapeDtypeStruct(x.shape, jnp.float32),
    )(x, den.reshape(-1, 1))


def _rowscale(x, cnt, eps, blk):
    nb = x.shape[0] // blk
    d = x.shape[1]

    def body(x_ref, d_ref, o_ref):
        o_ref[...] = x_ref[...] * (1.0 / jnp.maximum(d_ref[...], eps))

    return pl.pallas_call(
        body,
        grid=(nb,),
        in_specs=[
            pl.BlockSpec((blk, d), lambda i: (i, 0)),
            pl.BlockSpec((blk, 1), lambda i: (i, 0)),
        ],
        out_specs=pl.BlockSpec((blk, d), lambda i: (i, 0)),
        out_shape=jax.ShapeDtypeStruct(x.shape, jnp.float32),
    )(x, cnt.reshape(-1, 1))


def _final(g, wc, n_real, blk):
    nb = g.shape[0] // blk
    d = g.shape[1]
    d_out = wc.shape[1]

    def body(x_ref, w_ref, o_ref):
        i = pl.program_id(0)

        @pl.when(i == 0)
        def _():
            o_ref[...] = jnp.zeros_like(o_ref)

        s = jnp.sum(x_ref[...], axis=0, keepdims=True) * (1.0 / n_real)
        o_ref[...] += jnp.dot(s, w_ref[...], precision=lax.Precision.HIGHEST,
                               preferred_element_type=jnp.float32)

    return pl.pallas_call(
        body,
        grid=(nb,),
        in_specs=[
            pl.BlockSpec((blk, d), lambda i: (i, 0)),
            pl.BlockSpec((d, d_out), lambda i: (0, 0)),
        ],
        out_specs=pl.BlockSpec((1, d_out), lambda i: (0, 0)),
        out_shape=jax.ShapeDtypeStruct((1, d_out), jnp.float32),
    )(g, wc)


def _edge_meta(dst_sorted, cn, nchunk, slots):
    bounds = jnp.arange(nchunk + 1, dtype=jnp.int32) * cn
    pos = jnp.searchsorted(dst_sorted, bounds).astype(jnp.int32)
    starts = pos[:nchunk]
    ends = pos[1:]
    s8 = (starts // 8) * 8
    ng = (ends - s8 + (GB - 1)) // GB
    npad = slots * 32
    s8 = jnp.pad(s8, (0, npad - nchunk))
    ng = jnp.pad(ng, (0, npad - nchunk))
    s8 = jnp.pad(s8.reshape(slots, 32).T, ((0, 0), (0, 16 - slots)))
    ng = jnp.pad(ng.reshape(slots, 32).T, ((0, 0), (0, 16 - slots)))
    return jnp.concatenate([s8.reshape(-1), ng.reshape(-1)])


def kernel(patch_x, patch_edge_index, patch_edge_weight, patch_segment_ids,
           mesh_edge_index, W1, W2, W3, Wm1, Wm2, Wc):
    p_dst, p_src, p_w = lax.sort(
        (patch_edge_index[1], patch_edge_index[0], patch_edge_weight),
        num_keys=1)
    m_dst, m_src = lax.sort((mesh_edge_index[1], mesh_edge_index[0]),
                            num_keys=1)
    big = jnp.int32(2**30)
    p_src_p = jnp.pad(p_src, (0, EPAD_EXTRA))
    p_dst_p = jnp.pad(p_dst, (0, EPAD_EXTRA), constant_values=big)
    p_w_p = jnp.pad(p_w, (0, EPAD_EXTRA))
    m_src_p = jnp.pad(m_src, (0, EPAD_EXTRA))
    m_dst_p = jnp.pad(m_dst, (0, EPAD_EXTRA), constant_values=big)
    m_w_p = jnp.zeros((1,), jnp.float32)

    meta_p = _edge_meta(p_dst, CN_P, NCHUNK_P, SLOTS_P)
    meta_m = _edge_meta(m_dst, CN_M, NCHUNK_M, 1)
    ids_pad = jnp.pad(patch_segment_ids.astype(jnp.int32),
                      (0, PPAD - P_NODES), constant_values=MPAD)

    agg = _agg_build(HID, CN_P, NCHUNK_P, SLOTS_P, True)
    t = _matmul_plain(patch_x, W1, CN_P)
    num1, den1 = agg(t, p_src_p, p_dst_p, p_w_p, meta_p)
    h = _scale_relu(num1, den1, 1e-6, CN_P)

    num2, den2 = agg(h, p_src_p, p_dst_p, p_w_p, meta_p)
    h = _dense(num2, den2, W2, 1e-6, PPAD, CN_P)
    num3, den3 = agg(h, p_src_p, p_dst_p, p_w_p, meta_p)
    h = _dense(num3, den3, W3, 1e-6, PPAD, CN_P)

    meta_ro = _edge_meta(patch_segment_ids.astype(jnp.int32), CN_M, NCHUNK_M, 1)
    ro = _readout_build()
    sums, cnt = ro(h, ids_pad, meta_ro)
    readouts = _rowscale(sums, cnt, 1.0, CN_M)

    aggm = _agg_build(HID, CN_M, NCHUNK_M, 1, False)
    nm1, dm1 = aggm(readouts, m_src_p, m_dst_p, m_w_p, meta_m)
    g = _dense(nm1, dm1, Wm1, 1.0, MPAD, CN_M)
    nm2, dm2 = aggm(g, m_src_p, m_dst_p, m_w_p, meta_m)
    g = _dense(nm2, dm2, Wm2, 1.0, MPAD, CN_M)

    out = _final(g, Wc, float(N_MESH), CN_M)
    return out.reshape(OUT_FEATS)


# speedup vs baseline: 3.2860x; 1.0646x over previous
import functools

import jax
import jax.numpy as jnp
from jax import lax
from jax.experimental import pallas as pl
from jax.experimental.pallas import tpu as pltpu
from jax.experimental.pallas import tpu_sc as plsc

N_MESH = 10000
P_NODES = 200000
E_PATCH = 400000
E_MESH = 320000
IN_DIM = 32
HID = 128
OUT_FEATS = 16

NW = 32
CN_P = 512
NCHUNK_P = 392
PPAD = NCHUNK_P * CN_P
SLOTS_P = 13
CN_M = 320
NCHUNK_M = 32
MPAD = NCHUNK_M * CN_M
GB = 128
RG = 49
EPAD_EXTRA = 1024


def _agg_build(d_feat, cn, nchunk, slots, weighted):
    npad = nchunk * cn
    mesh = plsc.VectorSubcoreMesh(core_axis_name="c", subcore_axis_name="s")

    @functools.partial(
        pl.kernel,
        out_type=(
            jax.ShapeDtypeStruct((npad, d_feat), jnp.float32),
            jax.ShapeDtypeStruct((npad,), jnp.float32),
        ),
        mesh=mesh,
        compiler_params=pltpu.CompilerParams(needs_layout_passes=False),
        scratch_types=[
            pltpu.VMEM((16,), jnp.int32),
            pltpu.VMEM((16,), jnp.int32),
            pltpu.VMEM((2, GB), jnp.int32),
            pltpu.VMEM((2, GB), jnp.int32),
            pltpu.VMEM((2, GB), jnp.float32),
            pltpu.VMEM((2, GB, d_feat), jnp.float32),
            pltpu.VMEM((cn, d_feat), jnp.float32),
            pltpu.VMEM((cn,), jnp.float32),
            pltpu.SemaphoreType.DMA,
            pltpu.SemaphoreType.DMA,
            pltpu.SemaphoreType.DMA,
            pltpu.SemaphoreType.DMA,
        ],
    )
    def k(h_hbm, ss_hbm, ds_hbm, ws_hbm, meta_hbm, num_hbm, den_hbm,
          ms8_v, mng_v, src_v, dst_v, w_v, rows_v, acc_v, accd_v,
          sem0, sem1, semi0, semi1):
        cid = lax.axis_index("c")
        sid = lax.axis_index("s")
        wid = sid * 2 + cid
        lanes = lax.iota(jnp.int32, 16)
        zero16 = jnp.zeros((16,), jnp.float32)

        moff = pl.multiple_of(wid * 16, 8)
        pltpu.sync_copy(meta_hbm.at[pl.ds(moff, 16)], ms8_v)
        pltpu.sync_copy(meta_hbm.at[pl.ds(512 + moff, 16)], mng_v)

        def slot_body(slot, _):
            c = wid + 32 * slot
            ssplat = jnp.zeros((16,), jnp.int32) + slot

            @pl.when(c < nchunk)
            def _process():
                lo = c * cn
                hi = lo + cn
                s8 = pl.multiple_of(plsc.load_gather(ms8_v, [ssplat])[0], 8)
                ng = plsc.load_gather(mng_v, [ssplat])[0]

                def zrow(i, _):
                    def zcol(j, _):
                        acc_v[i, pl.ds(j * 16, 16)] = zero16
                        return 0
                    lax.fori_loop(0, d_feat // 16, zcol, 0, unroll=True)
                    return 0
                lax.fori_loop(0, cn, zrow, 0)

                def zden(i, _):
                    accd_v[pl.ds(i * 16, 16)] = zero16
                    return 0
                lax.fori_loop(0, cn // 16, zden, 0)

                sems = (sem0, sem1)
                semis = (semi0, semi1)

                def issue(g, p):
                    e0 = pl.multiple_of(s8 + g * GB, 8)
                    pltpu.sync_copy(ss_hbm.at[pl.ds(e0, GB)], src_v.at[p])
                    pltpu.async_copy(h_hbm.at[src_v.at[p]], rows_v.at[p],
                                     sems[p])
                    pltpu.async_copy(ds_hbm.at[pl.ds(e0, GB)], dst_v.at[p],
                                     semis[p])
                    if weighted:
                        pltpu.async_copy(ws_hbm.at[pl.ds(e0, GB)], w_v.at[p],
                                         semis[p])

                def compute(g, p):
                    e0 = pl.multiple_of(s8 + g * GB, 8)
                    pltpu.make_async_copy(ds_hbm.at[pl.ds(e0, GB)],
                                          dst_v.at[p], semis[p]).wait()
                    if weighted:
                        pltpu.make_async_copy(ws_hbm.at[pl.ds(e0, GB)],
                                              w_v.at[p], semis[p]).wait()
                    pltpu.make_async_copy(h_hbm.at[src_v.at[p]],
                                          rows_v.at[p], sems[p]).wait()

                    def batch(b, _):
                        dv = dst_v[p, pl.ds(b * 16, 16)]
                        m = jnp.logical_and(dv >= lo, dv < hi)
                        if weighted:
                            wv = w_v[p, pl.ds(b * 16, 16)]
                        else:
                            wv = zero16 + 1.0
                        wm = jnp.where(m, wv, 0.0)
                        dl = jnp.where(m, dv - lo, 0)
                        plsc.addupdate_scatter(accd_v, [dl], wm)

                        for j in range(16):
                            r = dl[j]
                            wj = wm[j]
                            e = b * 16 + j
                            for kk in range(d_feat // 16):
                                x = rows_v[p, e, pl.ds(kk * 16, 16)]
                                plsc.addupdate(
                                    acc_v.at[r, pl.ds(kk * 16, 16)], x * wj)
                        return 0
                    lax.fori_loop(0, GB // 16, batch, 0)

                @pl.when(ng > 0)
                def _prime():
                    issue(0, 0)

                def gpair(gp, _):
                    g0 = gp * 2

                    @pl.when(g0 + 1 < ng)
                    def _():
                        issue(g0 + 1, 1)
                    compute(g0, 0)

                    @pl.when(g0 + 1 < ng)
                    def _():
                        @pl.when(g0 + 2 < ng)
                        def _():
                            issue(g0 + 2, 0)
                        compute(g0 + 1, 1)
                    return 0
                lax.fori_loop(0, (ng + 1) // 2, gpair, 0)

                lo8 = pl.multiple_of(lo, 8)
                pltpu.sync_copy(acc_v, num_hbm.at[pl.ds(lo8, cn)])
                pltpu.sync_copy(accd_v, den_hbm.at[pl.ds(lo8, cn)])

            return 0

        lax.fori_loop(0, slots, slot_body, 0)

    return k


def _readout_build():
    mesh = plsc.VectorSubcoreMesh(core_axis_name="c", subcore_axis_name="s")

    @functools.partial(
        pl.kernel,
        out_type=(
            jax.ShapeDtypeStruct((MPAD, HID), jnp.float32),
            jax.ShapeDtypeStruct((MPAD,), jnp.float32),
        ),
        mesh=mesh,
        compiler_params=pltpu.CompilerParams(needs_layout_passes=False),
        scratch_types=[
            pltpu.VMEM((16,), jnp.int32),
            pltpu.VMEM((16,), jnp.int32),
            pltpu.VMEM((2, GB), jnp.int32),
            pltpu.VMEM((2, GB, HID), jnp.float32),
            pltpu.VMEM((CN_M, HID), jnp.float32),
            pltpu.VMEM((CN_M,), jnp.float32),
            pltpu.SemaphoreType.DMA,
            pltpu.SemaphoreType.DMA,
        ],
    )
    def k(h_hbm, ids_hbm, meta_hbm, sums_hbm, cnt_hbm,
          ms8_v, mng_v, ids_v, rows_v, acc_v, accd_v, sem0, sem1):
        cid = lax.axis_index("c")
        sid = lax.axis_index("s")
        wid = sid * 2 + cid
        lanes = lax.iota(jnp.int32, 16)
        zero16 = jnp.zeros((16,), jnp.float32)
        m_lo = pl.multiple_of(wid * CN_M, 8)
        m_hi = m_lo + CN_M
        moff = pl.multiple_of(wid * 16, 8)
        pltpu.sync_copy(meta_hbm.at[pl.ds(moff, 16)], ms8_v)
        pltpu.sync_copy(meta_hbm.at[pl.ds(512 + moff, 16)], mng_v)
        r0 = pl.multiple_of(ms8_v[...][0], 8)
        ng = mng_v[...][0]

        def zrow(i, _):
            def zcol(j, _):
                acc_v[i, pl.ds(j * 16, 16)] = zero16
                return 0
            lax.fori_loop(0, HID // 16, zcol, 0, unroll=True)
            return 0
        lax.fori_loop(0, CN_M, zrow, 0)

        def zden(i, _):
            accd_v[pl.ds(i * 16, 16)] = zero16
            return 0
        lax.fori_loop(0, CN_M // 16, zden, 0)

        sems = (sem0, sem1)

        def issue(g, p):
            e0 = pl.multiple_of(r0 + g * GB, 8)
            pltpu.async_copy(h_hbm.at[pl.ds(e0, GB)], rows_v.at[p], sems[p])
            pltpu.sync_copy(ids_hbm.at[pl.ds(e0, GB)], ids_v.at[p])

        def compute(g, p):
            e0 = pl.multiple_of(r0 + g * GB, 8)
            pltpu.make_async_copy(h_hbm.at[pl.ds(e0, GB)], rows_v.at[p],
                                  sems[p]).wait()

            def batch(b, _):
                iv = ids_v[p, pl.ds(b * 16, 16)]
                m = jnp.logical_and(iv >= m_lo, iv < m_hi)
                wm = jnp.where(m, 1.0, 0.0)
                dl = jnp.where(m, iv - m_lo, 0)
                plsc.addupdate_scatter(accd_v, [dl], wm)

                for j in range(16):
                    r = dl[j]
                    wj = wm[j]
                    e = b * 16 + j
                    for kk in range(HID // 16):
                        x = rows_v[p, e, pl.ds(kk * 16, 16)]
                        plsc.addupdate(acc_v.at[r, pl.ds(kk * 16, 16)], x * wj)
                return 0
            lax.fori_loop(0, GB // 16, batch, 0)

        @pl.when(ng > 0)
        def _prime():
            issue(0, 0)

        def gpair(gp, _):
            g0 = gp * 2

            @pl.when(g0 + 1 < ng)
            def _():
                issue(g0 + 1, 1)
            compute(g0, 0)

            @pl.when(g0 + 1 < ng)
            def _():
                @pl.when(g0 + 2 < ng)
                def _():
                    issue(g0 + 2, 0)
                compute(g0 + 1, 1)
            return 0
        lax.fori_loop(0, (ng + 1) // 2, gpair, 0)

        mlo8 = pl.multiple_of(m_lo, 8)
        pltpu.sync_copy(acc_v, sums_hbm.at[pl.ds(mlo8, CN_M)])
        pltpu.sync_copy(accd_v, cnt_hbm.at[pl.ds(mlo8, CN_M)])

    return k


def _dense(num, den, w, eps, rows_out, blk):
    nb = num.shape[0] // blk
    d_in = num.shape[1]
    d_out = w.shape[1]

    def body(x_ref, d_ref, w_ref, o_ref):
        r = 1.0 / jnp.maximum(d_ref[...], eps)
        x = x_ref[...] * r
        o_ref[...] = jnp.maximum(
            jnp.dot(x, w_ref[...], precision=lax.Precision.HIGHEST,
                    preferred_element_type=jnp.float32), 0.0)

    return pl.pallas_call(
        body,
        grid=(nb,),
        in_specs=[
            pl.BlockSpec((blk, d_in), lambda i: (i, 0)),
            pl.BlockSpec((blk, 1), lambda i: (i, 0)),
            pl.BlockSpec((d_in, d_out), lambda i: (0, 0)),
        ],
        out_specs=pl.BlockSpec((blk, d_out), lambda i: (i, 0)),
        out_shape=jax.ShapeDtypeStruct((rows_out, d_out), jnp.float32),
    )(num, den.reshape(-1, 1), w)


def _matmul_plain(x, w, blk):
    rows = x.shape[0]
    nb = pl.cdiv(rows, blk)
    d_in = x.shape[1]
    d_out = w.shape[1]

    def body(x_ref, w_ref, o_ref):
        o_ref[...] = jnp.dot(x_ref[...], w_ref[...],
                             precision=lax.Precision.HIGHEST,
                             preferred_element_type=jnp.float32)

    return pl.pallas_call(
        body,
        grid=(nb,),
        in_specs=[
            pl.BlockSpec((blk, d_in), lambda i: (i, 0)),
            pl.BlockSpec((d_in, d_out), lambda i: (0, 0)),
        ],
        out_specs=pl.BlockSpec((blk, d_out), lambda i: (i, 0)),
        out_shape=jax.ShapeDtypeStruct((rows, d_out), jnp.float32),
    )(x, w)


def _scale_relu(x, den, eps, blk):
    nb = x.shape[0] // blk
    d = x.shape[1]

    def body(x_ref, d_ref, o_ref):
        r = 1.0 / jnp.maximum(d_ref[...], eps)
        o_ref[...] = jnp.maximum(x_ref[...] * r, 0.0)

    return pl.pallas_call(
        body,
        grid=(nb,),
        in_specs=[
            pl.BlockSpec((blk, d), lambda i: (i, 0)),
            pl.BlockSpec((blk, 1), lambda i: (i, 0)),
        ],
        out_specs=pl.BlockSpec((blk, d), lambda i: (i, 0)),
        out_shape=jax.ShapeDtypeStruct(x.shape, jnp.float32),
    )(x, den.reshape(-1, 1))


def _rowscale(x, cnt, eps, blk):
    nb = x.shape[0] // blk
    d = x.shape[1]

    def body(x_ref, d_ref, o_ref):
        o_ref[...] = x_ref[...] * (1.0 / jnp.maximum(d_ref[...], eps))

    return pl.pallas_call(
        body,
        grid=(nb,),
        in_specs=[
            pl.BlockSpec((blk, d), lambda i: (i, 0)),
            pl.BlockSpec((blk, 1), lambda i: (i, 0)),
        ],
        out_specs=pl.BlockSpec((blk, d), lambda i: (i, 0)),
        out_shape=jax.ShapeDtypeStruct(x.shape, jnp.float32),
    )(x, cnt.reshape(-1, 1))


def _final(g, wc, n_real, blk):
    nb = g.shape[0] // blk
    d = g.shape[1]
    d_out = wc.shape[1]

    def body(x_ref, w_ref, o_ref):
        i = pl.program_id(0)

        @pl.when(i == 0)
        def _():
            o_ref[...] = jnp.zeros_like(o_ref)

        s = jnp.sum(x_ref[...], axis=0, keepdims=True) * (1.0 / n_real)
        o_ref[...] += jnp.dot(s, w_ref[...], precision=lax.Precision.HIGHEST,
                               preferred_element_type=jnp.float32)

    return pl.pallas_call(
        body,
        grid=(nb,),
        in_specs=[
            pl.BlockSpec((blk, d), lambda i: (i, 0)),
            pl.BlockSpec((d, d_out), lambda i: (0, 0)),
        ],
        out_specs=pl.BlockSpec((1, d_out), lambda i: (0, 0)),
        out_shape=jax.ShapeDtypeStruct((1, d_out), jnp.float32),
    )(g, wc)


def _edge_meta(dst_sorted, cn, nchunk, slots):
    bounds = jnp.arange(nchunk + 1, dtype=jnp.int32) * cn
    pos = jnp.searchsorted(dst_sorted, bounds).astype(jnp.int32)
    starts = pos[:nchunk]
    ends = pos[1:]
    s8 = (starts // 8) * 8
    ng = (ends - s8 + (GB - 1)) // GB
    npad = slots * 32
    s8 = jnp.pad(s8, (0, npad - nchunk))
    ng = jnp.pad(ng, (0, npad - nchunk))
    s8 = jnp.pad(s8.reshape(slots, 32).T, ((0, 0), (0, 16 - slots)))
    ng = jnp.pad(ng.reshape(slots, 32).T, ((0, 0), (0, 16 - slots)))
    return jnp.concatenate([s8.reshape(-1), ng.reshape(-1)])


def kernel(patch_x, patch_edge_index, patch_edge_weight, patch_segment_ids,
           mesh_edge_index, W1, W2, W3, Wm1, Wm2, Wc):
    p_dst, p_src, p_w = lax.sort(
        (patch_edge_index[1], patch_edge_index[0], patch_edge_weight),
        num_keys=1)
    m_dst, m_src = lax.sort((mesh_edge_index[1], mesh_edge_index[0]),
                            num_keys=1)
    big = jnp.int32(2**30)
    p_src_p = jnp.pad(p_src, (0, EPAD_EXTRA))
    p_dst_p = jnp.pad(p_dst, (0, EPAD_EXTRA), constant_values=big)
    p_w_p = jnp.pad(p_w, (0, EPAD_EXTRA))
    m_src_p = jnp.pad(m_src, (0, EPAD_EXTRA))
    m_dst_p = jnp.pad(m_dst, (0, EPAD_EXTRA), constant_values=big)
    m_w_p = jnp.zeros((1,), jnp.float32)

    meta_p = _edge_meta(p_dst, CN_P, NCHUNK_P, SLOTS_P)
    meta_m = _edge_meta(m_dst, CN_M, NCHUNK_M, 1)
    ids_pad = jnp.pad(patch_segment_ids.astype(jnp.int32),
                      (0, PPAD - P_NODES), constant_values=MPAD)

    agg = _agg_build(HID, CN_P, NCHUNK_P, SLOTS_P, True)
    t = _matmul_plain(patch_x, W1, CN_P)
    num1, den1 = agg(t, p_src_p, p_dst_p, p_w_p, meta_p)
    h = _scale_relu(num1, den1, 1e-6, CN_P)

    num2, den2 = agg(h, p_src_p, p_dst_p, p_w_p, meta_p)
    h = _dense(num2, den2, W2, 1e-6, PPAD, CN_P)
    num3, den3 = agg(h, p_src_p, p_dst_p, p_w_p, meta_p)
    h = _dense(num3, den3, W3, 1e-6, PPAD, CN_P)

    meta_ro = _edge_meta(patch_segment_ids.astype(jnp.int32), CN_M, NCHUNK_M, 1)
    ro = _readout_build()
    sums, cnt = ro(h, ids_pad, meta_ro)
    readouts = _rowscale(sums, cnt, 1.0, CN_M)

    aggm = _agg_build(HID, CN_M, NCHUNK_M, 1, False)
    nm1, dm1 = aggm(readouts, m_src_p, m_dst_p, m_w_p, meta_m)
    g = _dense(nm1, dm1, Wm1, 1.0, MPAD, CN_M)
    nm2, dm2 = aggm(g, m_src_p, m_dst_p, m_w_p, meta_m)
    g = _dense(nm2, dm2, Wm2, 1.0, MPAD, CN_M)

    out = _final(g, Wc, float(N_MESH), CN_M)
    return out.reshape(OUT_FEATS)
